# baseline (speedup 1.0000x reference)
import functools
import math

import jax
import jax.numpy as jnp
from jax import lax
from jax.experimental import pallas as pl
from jax.experimental.pallas import tpu as pltpu
from jax.experimental.pallas import tpu_sc as plsc

_NC = 2
_NS = 16
_LANES = 16
_NW = _NC * _NS

_KL = 2
_NBUF = 2


@functools.lru_cache(maxsize=None)
def _build_repack(v, d, scale):
    vp = v // 2
    full = v // 128
    nb_lo, rem = divmod(full, _NW)
    mesh = plsc.VectorSubcoreMesh(core_axis_name="c", subcore_axis_name="s")

    @functools.partial(
        pl.kernel,
        out_type=jax.ShapeDtypeStruct((vp, 2 * d), jnp.float32),
        mesh=mesh,
        scratch_types=[
            [pltpu.VMEM((d, 128), jnp.float32) for _ in range(_NBUF)],
            [pltpu.VMEM((64, 2 * d), jnp.float32) for _ in range(_NBUF)],
            pltpu.VMEM((64, d), jnp.float32),
            [pltpu.SemaphoreType.DMA for _ in range(_NBUF)],
            [pltpu.SemaphoreType.DMA for _ in range(_NBUF)],
        ],
        compiler_params=pltpu.CompilerParams(
            use_tc_tiling_on_sc=True, needs_layout_passes=False
        ),
    )
    def repack(tt_hbm, tail_hbm, out_hbm, tb, pb, tailb, gsem, osem):
        wid = lax.axis_index("s") * _NC + lax.axis_index("c")
        lo = wid * nb_lo + jnp.minimum(wid, rem)
        nblk = nb_lo + jnp.where(wid < rem, 1, 0)

        iota = lax.iota(jnp.int32, _LANES)
        toks = [iota + g * _LANES for g in range(8)]
        riv = [t >> 1 for t in toks]
        parcol = (iota & 1) * d

        def start_read(i, b):
            ib = lo + i
            pltpu.async_copy(
                tt_hbm.at[:, pl.ds(ib * 128, 128)], tb[b], gsem[b]
            )

        for b in range(_NBUF):
            @pl.when(b < nblk)
            def _():
                start_read(b, b)

        n_steps = nb_lo + _NBUF

        @pl.loop(0, n_steps, step=_NBUF)
        def _(i0):
            for b in range(_NBUF):
                i = i0 + b

                @pl.when(i < nblk)
                def _():
                    pltpu.make_async_copy(
                        tt_hbm.at[:, pl.ds(0, 128)], tb[b], gsem[b]
                    ).wait()

                    @pl.when(i >= _NBUF)
                    def _():
                        pltpu.make_async_copy(
                            pb[b], out_hbm.at[pl.ds(0, 64)], osem[b]
                        ).wait()

                    @pl.loop(0, d, unroll=8)
                    def _(c):
                        cvec = parcol + c
                        for g in range(8):
                            vals = tb[b][c, pl.ds(g * _LANES, _LANES)] * scale
                            plsc.store_scatter(pb[b], [riv[g], cvec], vals)

                    pltpu.async_copy(
                        pb[b], out_hbm.at[pl.ds((lo + i) * 64, 64)], osem[b]
                    )

                    @pl.when(i + _NBUF < nblk)
                    def _():
                        start_read(i + _NBUF, b)

        for b in range(_NBUF):
            @pl.when(b < jnp.minimum(nblk, _NBUF))
            def _():
                pltpu.make_async_copy(
                    pb[b], out_hbm.at[pl.ds(0, 64)], osem[b]
                ).wait()

        @pl.when(wid == 0)
        def _():
            pltpu.sync_copy(tail_hbm, tailb)

            @pl.loop(0, d, unroll=8)
            def _(c):
                cvec = jnp.full((_LANES,), c, jnp.int32)
                for g in range(4):
                    vals = plsc.load_gather(tailb, [toks[g], cvec]) * scale
                    plsc.store_scatter(
                        pb[0].at[pl.ds(0, 32)], [riv[g], parcol + c], vals
                    )

            pltpu.sync_copy(
                pb[0].at[pl.ds(0, 32)], out_hbm.at[pl.ds(full * 64, 32)]
            )

    return repack


@functools.lru_cache(maxsize=None)
def _build_gather(b_sz, l_sz, vp, d):
    n = b_sz * l_sz
    per_w = n // _NW
    assert per_w == 128 * l_sz and l_sz % (_KL * _NBUF) == 0
    cb_n = d // 8
    g_rows = 128 * _KL
    mesh = plsc.VectorSubcoreMesh(core_axis_name="c", subcore_axis_name="s")

    @functools.partial(
        pl.kernel,
        out_type=jax.ShapeDtypeStruct((l_sz, cb_n, _NW, 8, 128), jnp.float32),
        mesh=mesh,
        scratch_types=[
            pltpu.VMEM((per_w,), jnp.int32),
            [pltpu.VMEM((g_rows,), jnp.int32) for _ in range(_NBUF)],
            [pltpu.VMEM((g_rows,), jnp.int32) for _ in range(_NBUF)],
            [pltpu.VMEM((g_rows, 2 * d), jnp.float32) for _ in range(_NBUF)],
            [pltpu.VMEM((_KL, d, 128), jnp.float32) for _ in range(_NBUF)],
            [pltpu.SemaphoreType.DMA for _ in range(_NBUF)],
            [pltpu.SemaphoreType.DMA for _ in range(_NBUF)],
        ],
        compiler_params=pltpu.CompilerParams(
            use_tc_tiling_on_sc=False, needs_layout_passes=False
        ),
    )
    def gat(tp_hbm, idx_hbm, out_hbm, idx_v, cidx, par, rows, tbuf, gsem, osem):
        wid = lax.axis_index("s") * _NC + lax.axis_index("c")
        base = wid * per_w

        pltpu.sync_copy(idx_hbm.at[pl.ds(base, per_w)], idx_v)

        iota = lax.iota(jnp.int32, _LANES)
        toks = [iota + g * _LANES for g in range(8)]
        strides = [t * l_sz for t in toks]

        def build_and_gather(step, b):
            for j in range(_KL):
                l = step * _KL + j
                for g in range(8):
                    ids = plsc.load_gather(idx_v, [strides[g] + l])
                    o = j * 128 + g * _LANES
                    cidx[b][pl.ds(o, _LANES)] = ids >> 1
                    par[b][pl.ds(o, _LANES)] = (ids & 1) * d
            pltpu.async_copy(tp_hbm.at[cidx[b]], rows[b], gsem[b])

        for b in range(_NBUF):
            build_and_gather(b, b)

        n_chunks = l_sz // _KL

        @pl.loop(0, n_chunks, step=_NBUF)
        def _(s0):
            for b in range(_NBUF):
                step = s0 + b
                pltpu.make_async_copy(
                    tp_hbm.at[pl.ds(0, g_rows)], rows[b], gsem[b]
                ).wait()

                @pl.when(step >= _NBUF)
                def _():
                    for j in range(_KL):
                        for cb in range(cb_n):
                            pltpu.make_async_copy(
                                tbuf[b].at[0, pl.ds(cb * 8, 8)],
                                out_hbm.at[0, cb, wid],
                                osem[b],
                            ).wait()

                for j in range(_KL):
                    pvecs = [
                        par[b][pl.ds(j * 128 + g * _LANES, _LANES)]
                        for g in range(8)
                    ]
                    rowv = [toks[g] + j * 128 for g in range(8)]

                    @pl.loop(0, d, unroll=8)
                    def _(c):
                        for g in range(8):
                            vals = plsc.load_gather(
                                rows[b], [rowv[g], pvecs[g] + c]
                            )
                            tbuf[b][j, c, pl.ds(g * _LANES, _LANES)] = vals

                for j in range(_KL):
                    for cb in range(cb_n):
                        pltpu.async_copy(
                            tbuf[b].at[j, pl.ds(cb * 8, 8)],
                            out_hbm.at[step * _KL + j, cb, wid],
                            osem[b],
                        )

                @pl.when(step + _NBUF < n_chunks)
                def _():
                    build_and_gather(step + _NBUF, b)

        for b in range(_NBUF):
            for j in range(_KL):
                for cb in range(cb_n):
                    pltpu.make_async_copy(
                        tbuf[b].at[0, pl.ds(cb * 8, 8)],
                        out_hbm.at[0, cb, wid],
                        osem[b],
                    ).wait()

    return gat


def kernel(x, table):
    b_sz, l_sz = x.shape
    v, d = table.shape
    idx = x.reshape(-1).astype(jnp.int32)
    full = v // 128
    tail = lax.slice(table, (full * 128, 0), (v, d))
    packed = _build_repack(v, d, math.sqrt(d))(table.T, tail)
    out5 = _build_gather(b_sz, l_sz, v // 2, d)(packed, idx)
    return out5.transpose(2, 4, 0, 1, 3).reshape(b_sz, l_sz, d)


# speedup vs baseline: 2.1188x; 2.1188x over previous
import functools
import math

import jax
import jax.numpy as jnp
from jax import lax
from jax.experimental import pallas as pl
from jax.experimental.pallas import tpu as pltpu
from jax.experimental.pallas import tpu_sc as plsc

_NC = 2
_NS = 16
_LANES = 16
_NW = _NC * _NS

_NBUF = 4


@functools.lru_cache(maxsize=None)
def _build(b_sz, l_sz, v, d, scale):
    assert b_sz % (_NW * _NBUF) == 0 and d % _LANES == 0 and l_sz % 8 == 0
    rows_w = b_sz // _NW
    per_w = rows_w * l_sz
    mesh = plsc.VectorSubcoreMesh(core_axis_name="c", subcore_axis_name="s")

    @functools.partial(
        pl.kernel,
        out_type=jax.ShapeDtypeStruct((b_sz, l_sz, d), jnp.float32),
        mesh=mesh,
        scratch_types=[
            pltpu.VMEM((per_w,), jnp.int32),
            [pltpu.VMEM((l_sz, d), jnp.float32) for _ in range(_NBUF)],
            [pltpu.SemaphoreType.DMA for _ in range(_NBUF)],
            [pltpu.SemaphoreType.DMA for _ in range(_NBUF)],
        ],
        compiler_params=pltpu.CompilerParams(
            use_tc_tiling_on_sc=False, needs_layout_passes=False
        ),
    )
    def emb(table_hbm, idx_hbm, out_hbm, idx_v, rows, gsem, osem):
        wid = lax.axis_index("s") * _NC + lax.axis_index("c")
        base = wid * per_w
        row0 = wid * rows_w

        pltpu.sync_copy(idx_hbm.at[pl.ds(base, per_w)], idx_v)

        def start_gather(step, b):
            pltpu.async_copy(
                table_hbm.at[idx_v.at[pl.ds(step * l_sz, l_sz)]],
                rows[b],
                gsem[b],
            )

        for b in range(_NBUF):
            start_gather(b, b)

        @pl.loop(0, rows_w, step=_NBUF)
        def _(r0):
            for b in range(_NBUF):
                step = r0 + b
                pltpu.make_async_copy(
                    table_hbm.at[pl.ds(0, l_sz)], rows[b], gsem[b]
                ).wait()

                @pl.loop(0, l_sz, unroll=8)
                def _(i):
                    for j in range(d // _LANES):
                        sl = pl.ds(j * _LANES, _LANES)
                        rows[b][i, sl] = rows[b][i, sl] * scale

                out_slice = out_hbm.at[row0 + step]
                pltpu.async_copy(rows[b], out_slice, osem[b])

                @pl.when(step + _NBUF < rows_w)
                def _():
                    pltpu.make_async_copy(rows[b], out_slice, osem[b]).wait()
                    start_gather(step + _NBUF, b)

        for b in range(_NBUF):
            pltpu.make_async_copy(rows[b], out_hbm.at[0], osem[b]).wait()

    return emb


def kernel(x, table):
    b_sz, l_sz = x.shape
    v, d = table.shape
    idx = x.reshape(-1).astype(jnp.int32)
    return _build(b_sz, l_sz, v, d, math.sqrt(d))(table, idx)


# speedup vs baseline: 2.1242x; 1.0025x over previous
import functools
import math

import jax
import jax.numpy as jnp
from jax import lax
from jax.experimental import pallas as pl
from jax.experimental.pallas import tpu as pltpu
from jax.experimental.pallas import tpu_sc as plsc

_NC = 2
_NS = 16
_LANES = 16
_NW = _NC * _NS

_NBUF = 4


@functools.lru_cache(maxsize=None)
def _build(b_sz, l_sz, v, d, scale):
    assert b_sz % (_NW * _NBUF) == 0 and d % _LANES == 0 and l_sz % 8 == 0
    rows_w = b_sz // _NW
    per_w = rows_w * l_sz
    mesh = plsc.VectorSubcoreMesh(core_axis_name="c", subcore_axis_name="s")

    @functools.partial(
        pl.kernel,
        out_type=jax.ShapeDtypeStruct((b_sz, l_sz, d), jnp.float32),
        mesh=mesh,
        scratch_types=[
            pltpu.VMEM((rows_w, l_sz), jnp.int32),
            [pltpu.VMEM((l_sz, d), jnp.float32) for _ in range(_NBUF)],
            [pltpu.SemaphoreType.DMA for _ in range(_NBUF)],
            [pltpu.SemaphoreType.DMA for _ in range(_NBUF)],
        ],
        compiler_params=pltpu.CompilerParams(
            use_tc_tiling_on_sc=False, needs_layout_passes=False
        ),
    )
    def emb(table_hbm, idx_hbm, out_hbm, idx_v, rows, gsem, osem):
        wid = lax.axis_index("s") * _NC + lax.axis_index("c")
        row0 = wid * rows_w

        pltpu.sync_copy(idx_hbm.at[pl.ds(row0, rows_w)], idx_v)

        def start_gather(step, b):
            pltpu.async_copy(
                table_hbm.at[idx_v.at[step]],
                rows[b],
                gsem[b],
            )

        for b in range(_NBUF):
            start_gather(b, b)

        @pl.loop(0, rows_w, step=_NBUF)
        def _(r0):
            for b in range(_NBUF):
                step = r0 + b
                pltpu.make_async_copy(
                    table_hbm.at[pl.ds(0, l_sz)], rows[b], gsem[b]
                ).wait()

                @pl.loop(0, l_sz, unroll=8)
                def _(i):
                    for j in range(d // _LANES):
                        sl = pl.ds(j * _LANES, _LANES)
                        rows[b][i, sl] = rows[b][i, sl] * scale

                out_slice = out_hbm.at[row0 + step]
                pltpu.async_copy(rows[b], out_slice, osem[b])

                @pl.when(step + _NBUF < rows_w)
                def _():
                    pltpu.make_async_copy(rows[b], out_slice, osem[b]).wait()
                    start_gather(step + _NBUF, b)

        for b in range(_NBUF):
            pltpu.make_async_copy(rows[b], out_hbm.at[0], osem[b]).wait()

    return emb


def kernel(x, table):
    b_sz, l_sz = x.shape
    v, d = table.shape
    return _build(b_sz, l_sz, v, d, math.sqrt(d))(table, x.astype(jnp.int32))
